# Initial kernel scaffold; baseline (speedup 1.0000x reference)
import jax
import jax.numpy as jnp
from jax.experimental import pallas as pl


def kernel(x, edge_index_0, edge_index_1, embed_table, W_0_0, b_0_0, W_0_1, b_0_1, attW1_0, attb1_0, attW2_0, W_1_0, b_1_0, W_1_1, b_1_1, attW1_1, attb1_1, attW2_1):
    raise NotImplementedError("write your pallas kernel here")


# speedup vs baseline: 11.9167x; 11.9167x over previous
import functools

import jax
import jax.numpy as jnp
from jax import lax
from jax.experimental import pallas as pl
from jax.experimental.pallas import tpu as pltpu
from jax.experimental.pallas import tpu_sc as plsc

N = 10000
D = 128
E = 320000

NC = 2
NS = 16
CH = 128
NCHUNK = E // CH
CHUNKS_PER_TILE = -(-NCHUNK // NS)
RCH = 80
NRCH = N // RCH
RCH_PER_TILE = -(-NRCH // NS)
RCH_PER_WORKER = -(-NRCH // (NC * NS))

_mesh = plsc.VectorSubcoreMesh(core_axis_name="c", subcore_axis_name="s")


def _zero_rows(rows, nrows):
  z16 = jnp.zeros((16,), jnp.float32)
  def body(r, _):
    for j in range(8):
      rows[r, pl.ds(j * 16, 16)] = z16
    return 0
  lax.fori_loop(0, nrows, body, 0)


@functools.partial(
    pl.kernel,
    out_type=(
        jax.ShapeDtypeStruct((N, D), jnp.float32),
        jax.ShapeDtypeStruct((N,), jnp.float32),
        jax.ShapeDtypeStruct((N,), jnp.float32),
    ),
    mesh=_mesh,
    scratch_types=[
        pltpu.VMEM((RCH,), jnp.int32),
        pltpu.VMEM((RCH, D), jnp.float32),
        pltpu.VMEM((CH,), jnp.int32),
        pltpu.VMEM((CH,), jnp.float32),
        pltpu.VMEM((RCH,), jnp.float32),
        pltpu.VMEM_SHARED((N,), jnp.float32),
        pltpu.SemaphoreType.DMA,
    ],
)
def _sc_embed_deg(x_hbm, emb_hbm, dst0_hbm, dst1_hbm,
                  h_hbm, cnt0_hbm, cnt1_hbm,
                  xidx, erows, didx, ones_v, stage, acc1, sem):
  c = lax.axis_index("c")
  s = lax.axis_index("s")
  w = c * NS + s

  o16 = jnp.ones((16,), jnp.float32)
  for j in range(CH // 16):
    ones_v[pl.ds(j * 16, 16)] = o16
  z16 = jnp.zeros((16,), jnp.float32)
  for j in range(RCH // 16):
    stage[pl.ds(j * 16, 16)] = z16

  def zbody(k, _):
    ch = s + NS * k
    @pl.when(ch < NRCH)
    def _():
      off = pl.multiple_of(ch * RCH, 8)
      pltpu.sync_copy(stage, acc1.at[pl.ds(off, RCH)])
    return 0
  lax.fori_loop(0, RCH_PER_TILE, zbody, 0)

  def ebody(k, _):
    ch = w + NC * NS * k
    @pl.when(ch < NRCH)
    def _():
      off = pl.multiple_of(ch * RCH, 8)
      pltpu.sync_copy(x_hbm.at[pl.ds(off, RCH)], xidx)
      pltpu.async_copy(emb_hbm.at[xidx], erows, sem).wait()
      pltpu.sync_copy(erows, h_hbm.at[pl.ds(off, RCH)])
    return 0
  lax.fori_loop(0, RCH_PER_WORKER, ebody, 0)

  plsc.subcore_barrier()

  def dbody(dst_hbm):
    def body(k, _):
      ch = s + NS * k
      @pl.when(ch < NCHUNK)
      def _():
        off = pl.multiple_of(ch * CH, 8)
        pltpu.sync_copy(dst_hbm.at[pl.ds(off, CH)], didx)
        pltpu.sync_copy(ones_v, acc1.at[didx], add=True)
      return 0
    lax.fori_loop(0, CHUNKS_PER_TILE, body, 0)

  @pl.when(c == 0)
  def _():
    dbody(dst0_hbm)
  @pl.when(c == 1)
  def _():
    dbody(dst1_hbm)

  plsc.subcore_barrier()

  def obody(cnt_hbm):
    def body(k, _):
      ch = s + NS * k
      @pl.when(ch < NRCH)
      def _():
        off = pl.multiple_of(ch * RCH, 8)
        pltpu.sync_copy(acc1.at[pl.ds(off, RCH)], stage)
        pltpu.sync_copy(stage, cnt_hbm.at[pl.ds(off, RCH)])
      return 0
    lax.fori_loop(0, RCH_PER_TILE, body, 0)

  @pl.when(c == 0)
  def _():
    obody(cnt0_hbm)
  @pl.when(c == 1)
  def _():
    obody(cnt1_hbm)


G = 8
NKP = 160
NGRP = NKP // G
EPAD = NS * NKP


@functools.partial(
    pl.kernel,
    out_type=(
        jax.ShapeDtypeStruct((N, D), jnp.float32),
        jax.ShapeDtypeStruct((N, D), jnp.float32),
    ),
    mesh=_mesh,
    scratch_types=[
        pltpu.VMEM((G, CH), jnp.int32),
        pltpu.VMEM((G, CH), jnp.int32),
        pltpu.VMEM((2, CH, D), jnp.float32),
        pltpu.VMEM_SHARED((N, D), jnp.float32),
        pltpu.SemaphoreType.DMA,
        pltpu.SemaphoreType.DMA,
    ],
)
def _sc_edge_agg(y0_hbm, y1_hbm, src0_hbm, dst0_hbm, src1_hbm, dst1_hbm,
                 agg0_hbm, agg1_hbm,
                 sidxg, didxg, rows, acc, sem0, sem1):
  c = lax.axis_index("c")
  s = lax.axis_index("s")
  sems = (sem0, sem1)

  z16 = jnp.zeros((16,), jnp.float32)
  def zrow(r, _):
    for j in range(D // 16):
      rows[0, r, pl.ds(j * 16, 16)] = z16
    return 0
  lax.fori_loop(0, RCH, zrow, 0)

  def zbody(k, _):
    ch = s + NS * k
    @pl.when(ch < NRCH)
    def _():
      off = pl.multiple_of(ch * RCH, 8)
      pltpu.sync_copy(rows.at[0, pl.ds(0, RCH)], acc.at[pl.ds(off, RCH)])
    return 0
  lax.fori_loop(0, RCH_PER_TILE, zbody, 0)

  plsc.subcore_barrier()

  def ebody(y_hbm, srcR, dstR):
    start = s * NKP
    def gbody(g, _):
      base = start + g * G
      pltpu.sync_copy(srcR.at[pl.ds(base, G)], sidxg)
      pltpu.sync_copy(dstR.at[pl.ds(base, G)], didxg)
      descs = [None] * G
      descs[0] = pltpu.async_copy(y_hbm.at[sidxg.at[0]], rows.at[0], sems[0])
      for j in range(G):
        if j + 1 < G:
          descs[j + 1] = pltpu.async_copy(
              y_hbm.at[sidxg.at[j + 1]], rows.at[(j + 1) % 2],
              sems[(j + 1) % 2])
        descs[j].wait()
        @pl.when(base + j < NCHUNK)
        def _():
          pltpu.sync_copy(rows.at[j % 2], acc.at[didxg.at[j]], add=True)
      return 0
    lax.fori_loop(0, NGRP, gbody, 0)

  @pl.when(c == 0)
  def _():
    ebody(y0_hbm, src0_hbm, dst0_hbm)
  @pl.when(c == 1)
  def _():
    ebody(y1_hbm, src1_hbm, dst1_hbm)

  plsc.subcore_barrier()

  def obody(agg_hbm):
    def body(k, _):
      ch = s + NS * k
      @pl.when(ch < NRCH)
      def _():
        off = pl.multiple_of(ch * RCH, 8)
        pltpu.sync_copy(acc.at[pl.ds(off, RCH)], rows.at[0, pl.ds(0, RCH)])
        pltpu.sync_copy(rows.at[0, pl.ds(0, RCH)], agg_hbm.at[pl.ds(off, RCH)])
      return 0
    lax.fori_loop(0, RCH_PER_TILE, body, 0)

  @pl.when(c == 0)
  def _():
    obody(agg0_hbm)
  @pl.when(c == 1)
  def _():
    obody(agg1_hbm)


BLK = 1000

def _mm(a, w):
  return lax.dot_general(a, w, (((1,), (1,)), ((), ())),
                         preferred_element_type=jnp.float32)


def _tc_pre_body(h_ref, cnt0_ref, cnt1_ref, w00_ref, w01_ref,
                 y0_ref, y1_ref, dinv0_ref, dinv1_ref):
  h = h_ref[...]
  dinv0 = lax.rsqrt(cnt0_ref[...] + 1.0)
  dinv1 = lax.rsqrt(cnt1_ref[...] + 1.0)
  dinv0_ref[...] = dinv0
  dinv1_ref[...] = dinv1
  y0_ref[...] = _mm(h, w00_ref[...]) * dinv0
  y1_ref[...] = _mm(h, w01_ref[...]) * dinv1


def _attention(e0, e1, aw1, ab1, aw2):
  t0 = jnp.tanh(_mm(e0, aw1) + ab1)
  t1 = jnp.tanh(_mm(e1, aw1) + ab1)
  w0 = _mm(t0, aw2)
  w1 = _mm(t1, aw2)
  m = jnp.maximum(w0, w1)
  s0 = jnp.exp(w0 - m)
  s1 = jnp.exp(w1 - m)
  return (s0 * e0 + s1 * e1) / (s0 + s1)


def _tc_mid_body(agg0_ref, agg1_ref, y0_ref, y1_ref, dinv0_ref, dinv1_ref,
                 b00_ref, b01_ref, aw1_ref, ab1_ref, aw2_ref,
                 w10_ref, w11_ref,
                 yn0_ref, yn1_ref):
  dinv0 = dinv0_ref[...]
  dinv1 = dinv1_ref[...]
  e0 = (agg0_ref[...] + y0_ref[...]) * dinv0 + b00_ref[...]
  e1 = (agg1_ref[...] + y1_ref[...]) * dinv1 + b01_ref[...]
  h1 = _attention(e0, e1, aw1_ref[...], ab1_ref[...], aw2_ref[...])
  yn0_ref[...] = _mm(h1, w10_ref[...]) * dinv0
  yn1_ref[...] = _mm(h1, w11_ref[...]) * dinv1


def _tc_post_body(agg0_ref, agg1_ref, y0_ref, y1_ref, dinv0_ref, dinv1_ref,
                  b10_ref, b11_ref, aw1_ref, ab1_ref, aw2_ref,
                  out_ref):
  e0 = (agg0_ref[...] + y0_ref[...]) * dinv0_ref[...] + b10_ref[...]
  e1 = (agg1_ref[...] + y1_ref[...]) * dinv1_ref[...] + b11_ref[...]
  h2 = _attention(e0, e1, aw1_ref[...], ab1_ref[...], aw2_ref[...])
  m = jnp.max(h2, axis=1, keepdims=True)
  lse = m + jnp.log(jnp.sum(jnp.exp(h2 - m), axis=1, keepdims=True))
  out_ref[...] = h2 - lse


def _row_spec():
  return pl.BlockSpec((BLK, D), lambda i: (i, 0))


def _col_spec():
  return pl.BlockSpec((BLK, 1), lambda i: (i, 0))


def _full_spec(shape):
  return pl.BlockSpec(shape, lambda i: tuple(0 for _ in shape))


def _tc_pre(h, cnt0, cnt1, w00, w01):
  grid = (N // BLK,)
  return pl.pallas_call(
      _tc_pre_body,
      grid=grid,
      in_specs=[_row_spec(), _col_spec(), _col_spec(),
                _full_spec((D, D)), _full_spec((D, D))],
      out_specs=(_row_spec(), _row_spec(), _col_spec(), _col_spec()),
      out_shape=(
          jax.ShapeDtypeStruct((N, D), jnp.float32),
          jax.ShapeDtypeStruct((N, D), jnp.float32),
          jax.ShapeDtypeStruct((N, 1), jnp.float32),
          jax.ShapeDtypeStruct((N, 1), jnp.float32),
      ),
  )(h, cnt0, cnt1, w00, w01)


def _tc_mid(agg0, agg1, y0, y1, dinv0, dinv1, b00, b01, aw1, ab1, aw2,
            w10, w11):
  grid = (N // BLK,)
  return pl.pallas_call(
      _tc_mid_body,
      grid=grid,
      in_specs=[_row_spec(), _row_spec(), _row_spec(), _row_spec(),
                _col_spec(), _col_spec(),
                _full_spec((1, D)), _full_spec((1, D)),
                _full_spec((D, D)), _full_spec((1, D)), _full_spec((1, D)),
                _full_spec((D, D)), _full_spec((D, D))],
      out_specs=(_row_spec(), _row_spec()),
      out_shape=(
          jax.ShapeDtypeStruct((N, D), jnp.float32),
          jax.ShapeDtypeStruct((N, D), jnp.float32),
      ),
  )(agg0, agg1, y0, y1, dinv0, dinv1, b00, b01, aw1, ab1, aw2, w10, w11)


def _tc_post(agg0, agg1, y0, y1, dinv0, dinv1, b10, b11, aw1, ab1, aw2):
  grid = (N // BLK,)
  return pl.pallas_call(
      _tc_post_body,
      grid=grid,
      in_specs=[_row_spec(), _row_spec(), _row_spec(), _row_spec(),
                _col_spec(), _col_spec(),
                _full_spec((1, D)), _full_spec((1, D)),
                _full_spec((D, D)), _full_spec((1, D)), _full_spec((1, D))],
      out_specs=_row_spec(),
      out_shape=jax.ShapeDtypeStruct((N, D), jnp.float32),
  )(agg0, agg1, y0, y1, dinv0, dinv1, b10, b11, aw1, ab1, aw2)


def kernel(x, edge_index_0, edge_index_1, embed_table,
           W_0_0, b_0_0, W_0_1, b_0_1, attW1_0, attb1_0, attW2_0,
           W_1_0, b_1_0, W_1_1, b_1_1, attW1_1, attb1_1, attW2_1):
  s0, d0 = edge_index_0[0], edge_index_0[1]
  s1, d1 = edge_index_1[0], edge_index_1[1]

  def _chunked(v):
    return jnp.pad(v.reshape(NCHUNK, CH), ((0, EPAD - NCHUNK), (0, 0)))

  s0c, d0c = _chunked(s0), _chunked(d0)
  s1c, d1c = _chunked(s1), _chunked(d1)

  h, cnt0, cnt1 = _sc_embed_deg(x, embed_table, d0, d1)

  y0, y1, dinv0, dinv1 = _tc_pre(
      h, cnt0.reshape(N, 1), cnt1.reshape(N, 1), W_0_0, W_0_1)

  agg0, agg1 = _sc_edge_agg(y0, y1, s0c, d0c, s1c, d1c)

  yn0, yn1 = _tc_mid(agg0, agg1, y0, y1, dinv0, dinv1,
                     b_0_0.reshape(1, D), b_0_1.reshape(1, D),
                     attW1_0, attb1_0.reshape(1, D), attW2_0,
                     W_1_0, W_1_1)

  agg0b, agg1b = _sc_edge_agg(yn0, yn1, s0c, d0c, s1c, d1c)

  return _tc_post(agg0b, agg1b, yn0, yn1, dinv0, dinv1,
                  b_1_0.reshape(1, D), b_1_1.reshape(1, D),
                  attW1_1, attb1_1.reshape(1, D), attW2_1)
